# Initial kernel scaffold; baseline (speedup 1.0000x reference)
import jax
import jax.numpy as jnp
from jax.experimental import pallas as pl


def kernel(rays_o, rays_d, rays_d_norm, ray_indices, t_nears, t_fars, table, beta):
    raise NotImplementedError("write your pallas kernel here")


# speedup vs baseline: 7.5614x; 7.5614x over previous
import functools

import jax
import jax.numpy as jnp
from jax import lax
from jax.experimental import pallas as pl
from jax.experimental.pallas import tpu as pltpu
from jax.experimental.pallas import tpu_sc as plsc

N_RAYS = 4096
TOTAL = 131072
NUM_EMB = 120000
VOXEL = 0.02
MIN_BETA = VOXEL

_NC = 2
_NS = 16
_NW = _NC * _NS

_P1 = 2654435761
_P2 = 805459861


def _make_sc_gather(V, D, B, K=16):
  IB = B // 128
  per_w = IB // _NW
  n_out = per_w // K
  assert IB % _NW == 0 and per_w % K == 0
  mesh = plsc.VectorSubcoreMesh(core_axis_name="c", subcore_axis_name="s")

  @functools.partial(
      pl.kernel,
      mesh=mesh,
      out_type=jax.ShapeDtypeStruct((B, D), jnp.float32),
      compiler_params=pltpu.CompilerParams(use_tc_tiling_on_sc=False),
      scratch_types=[
          pltpu.VMEM((K, 128), jnp.int32),
          pltpu.VMEM((K * 128, D), jnp.float32),
          pltpu.SemaphoreType.DMA,
      ],
  )
  def gather_kernel(table_hbm, idx_hbm, out_hbm, idx_v, rows_v, sem):
    wid = lax.axis_index("s") * _NC + lax.axis_index("c")
    row0 = wid * per_w

    def body(g, carry):
      r = row0 + g * K
      pltpu.sync_copy(idx_hbm.at[pl.ds(r, K)], idx_v)
      copies = []
      for j in range(K):
        copies.append(
            pltpu.async_copy(
                table_hbm.at[idx_v.at[j]],
                rows_v.at[pl.ds(j * 128, 128)],
                sem,
            ))
      for c in copies:
        c.wait()
      pltpu.sync_copy(rows_v, out_hbm.at[pl.ds(r * 128, K * 128)])
      return carry

    lax.fori_loop(0, n_out, body, 0)

  return gather_kernel


_gather_rays = _make_sc_gather(N_RAYS, 8, TOTAL)
_gather_emb = _make_sc_gather(NUM_EMB, 8, TOTAL * 8)

_RB = 2048
_RBF = 512


def _hash_body(rr_ref, tn_ref, tf_ref, idx_ref, frac_ref):
  rr = rr_ref[...]
  tm = 0.5 * (tn_ref[...] + tf_ref[...])
  fr = []
  x0i = []
  for d in range(3):
    xd = rr[:, d:d + 1] + tm * rr[:, d + 3:d + 4]
    xs = xd / jnp.float32(VOXEL)
    x0 = jnp.floor(xs)
    fr.append(xs - x0)
    x0i.append(x0.astype(jnp.int32))
  frac_ref[...] = jnp.concatenate(fr, axis=1)
  cols = []
  for i in (0, 1):
    for j in (0, 1):
      for k in (0, 1):
        ux = (x0i[0] + i).astype(jnp.uint32)
        uy = (x0i[1] + j).astype(jnp.uint32) * jnp.uint32(_P1)
        uz = (x0i[2] + k).astype(jnp.uint32) * jnp.uint32(_P2)
        h = (ux ^ uy ^ uz) % jnp.uint32(NUM_EMB)
        cols.append(h.astype(jnp.int32))
  idx_ref[...] = jnp.concatenate(cols, axis=1)


def _field_body(emb_ref, frac_ref, tn_ref, tf_ref, beta_ref,
                rgb_ref, grads_ref, sdt_ref):
  emb = emb_ref[...]
  frac = frac_ref[...]
  fx = frac[:, 0:1]
  fy = frac[:, 1:2]
  fz = frac[:, 2:3]
  sdf = jnp.zeros_like(fx)
  rgb = jnp.zeros((emb.shape[0], 3), jnp.float32)
  gx = jnp.zeros_like(fx)
  gy = jnp.zeros_like(fx)
  gz = jnp.zeros_like(fx)
  c = 0
  for i in (0, 1):
    for j in (0, 1):
      for k in (0, 1):
        tx = fx if i else 1.0 - fx
        ty = fy if j else 1.0 - fy
        tz = fz if k else 1.0 - fz
        w = tx * ty * tz
        sdf_c = emb[:, 8 * c:8 * c + 1]
        rgb_c = emb[:, 8 * c + 1:8 * c + 4]
        sdf = sdf + w * sdf_c
        rgb = rgb + w * rgb_c
        sx = 1.0 if i else -1.0
        sy = 1.0 if j else -1.0
        sz = 1.0 if k else -1.0
        gx = gx + sdf_c * (sx * ty * tz)
        gy = gy + sdf_c * (tx * sy * tz)
        gz = gz + sdf_c * (tx * ty * sz)
        c += 1
  inv_cell = 1.0 / jnp.float32(VOXEL)
  gx = gx * inv_cell
  gy = gy * inv_cell
  gz = gz * inv_cell
  grads_ref[...] = jnp.concatenate([gx, gy, gz], axis=1)
  beta_t = MIN_BETA + jnp.abs(beta_ref[0, 0])
  alpha = 1.0 / beta_t
  sig = alpha * (0.5 + 0.5 * jnp.sign(sdf) *
                 (jnp.exp(-jnp.abs(sdf) / beta_t) - 1.0))
  sdt_ref[...] = sig * (tf_ref[...] - tn_ref[...])
  rgb_ref[...] = rgb


def _hash_stage(ray_rows, t_nears, t_fars):
  grid = TOTAL // _RB
  return pl.pallas_call(
      _hash_body,
      grid=(grid,),
      in_specs=[
          pl.BlockSpec((_RB, 8), lambda i: (i, 0)),
          pl.BlockSpec((_RB, 1), lambda i: (i, 0)),
          pl.BlockSpec((_RB, 1), lambda i: (i, 0)),
      ],
      out_specs=[
          pl.BlockSpec((_RB, 8), lambda i: (i, 0)),
          pl.BlockSpec((_RB, 3), lambda i: (i, 0)),
      ],
      out_shape=[
          jax.ShapeDtypeStruct((TOTAL, 8), jnp.int32),
          jax.ShapeDtypeStruct((TOTAL, 3), jnp.float32),
      ],
  )(ray_rows, t_nears, t_fars)


def _field_stage(emb, frac, t_nears, t_fars, beta2d):
  grid = TOTAL // _RBF
  return pl.pallas_call(
      _field_body,
      grid=(grid,),
      in_specs=[
          pl.BlockSpec((_RBF, 64), lambda i: (i, 0)),
          pl.BlockSpec((_RBF, 3), lambda i: (i, 0)),
          pl.BlockSpec((_RBF, 1), lambda i: (i, 0)),
          pl.BlockSpec((_RBF, 1), lambda i: (i, 0)),
          pl.BlockSpec((1, 1), lambda i: (0, 0)),
      ],
      out_specs=[
          pl.BlockSpec((_RBF, 3), lambda i: (i, 0)),
          pl.BlockSpec((_RBF, 3), lambda i: (i, 0)),
          pl.BlockSpec((_RBF, 1), lambda i: (i, 0)),
      ],
      out_shape=[
          jax.ShapeDtypeStruct((TOTAL, 3), jnp.float32),
          jax.ShapeDtypeStruct((TOTAL, 3), jnp.float32),
          jax.ShapeDtypeStruct((TOTAL, 1), jnp.float32),
      ],
  )(emb, frac, t_nears, t_fars, beta2d)


@jax.jit
def _kernel_impl(rays_o, rays_d, rays_d_norm, ray_indices, t_nears, t_fars,
                 table, beta):
  rays_cat = jnp.concatenate(
      [rays_o, rays_d, jnp.zeros((N_RAYS, 2), jnp.float32)], axis=1)
  ridx2d = ray_indices.reshape(TOTAL // 128, 128)
  ray_rows = _gather_rays(rays_cat, ridx2d)

  idx, frac = _hash_stage(ray_rows, t_nears, t_fars)

  table_pad = jnp.concatenate(
      [table, jnp.zeros((NUM_EMB, 3), jnp.float32)], axis=1)
  emb_rows = _gather_emb(table_pad, idx.reshape(TOTAL * 8 // 128, 128))
  emb = emb_rows.reshape(TOTAL, 64)

  rgb, sdf_grads, sdt2 = _field_stage(emb, frac, t_nears, t_fars,
                                      beta.reshape(1, 1))

  gnorm = jnp.sqrt(jnp.sum(sdf_grads * sdf_grads, axis=1, keepdims=True))
  normals = sdf_grads / jnp.maximum(gnorm, 1e-12)

  sdt = sdt2[:, 0]
  cum = jnp.cumsum(sdt)
  excl = cum - sdt
  seg_start = jnp.concatenate(
      [jnp.ones((1,), bool), ray_indices[1:] != ray_indices[:-1]])
  base = lax.cummax(jnp.where(seg_start, excl, -jnp.inf))
  trans = jnp.exp(-(excl - base))
  weights = (1.0 - jnp.exp(-sdt)) * trans

  t_mid = 0.5 * (t_nears + t_fars)
  w1 = weights[:, None]
  vals = jnp.concatenate([w1 * rgb, w1 * t_mid, w1 * normals, w1], axis=1)
  seg = jax.ops.segment_sum(vals, ray_indices, num_segments=N_RAYS,
                            indices_are_sorted=True)
  rendered_rgb = seg[:, 0:3]
  rendered_depth = seg[:, 3:4] / rays_d_norm
  rendered_normals = seg[:, 4:7]
  accumulated_weights = seg[:, 7:8]
  return (rendered_rgb, rendered_depth, rendered_normals,
          accumulated_weights, sdf_grads)


def kernel(rays_o, rays_d, rays_d_norm, ray_indices, t_nears, t_fars, table,
           beta):
  return _kernel_impl(rays_o, rays_d, rays_d_norm, ray_indices, t_nears,
                      t_fars, table, beta)


# speedup vs baseline: 13.9595x; 1.8461x over previous
import functools

import jax
import jax.numpy as jnp
from jax import lax
from jax.experimental import pallas as pl
from jax.experimental.pallas import tpu as pltpu
from jax.experimental.pallas import tpu_sc as plsc

N_RAYS = 4096
TOTAL = 131072
NUM_EMB = 120000
VOXEL = 0.02
MIN_BETA = VOXEL

_NC = 2
_NS = 16
_NW = _NC * _NS

_P1 = 2654435761
_P2 = 805459861


def _make_sc_gather(V, D, B, K=16):
  IB = B // 128
  per_w = IB // _NW
  n_out = per_w // K
  assert IB % _NW == 0 and per_w % K == 0
  mesh = plsc.VectorSubcoreMesh(core_axis_name="c", subcore_axis_name="s")

  @functools.partial(
      pl.kernel,
      mesh=mesh,
      out_type=jax.ShapeDtypeStruct((B, D), jnp.float32),
      compiler_params=pltpu.CompilerParams(use_tc_tiling_on_sc=False),
      scratch_types=[
          pltpu.VMEM((K, 128), jnp.int32),
          pltpu.VMEM((K * 128, D), jnp.float32),
          pltpu.SemaphoreType.DMA,
      ],
  )
  def gather_kernel(table_hbm, idx_hbm, out_hbm, idx_v, rows_v, sem):
    wid = lax.axis_index("s") * _NC + lax.axis_index("c")
    row0 = wid * per_w

    def body(g, carry):
      r = row0 + g * K
      pltpu.sync_copy(idx_hbm.at[pl.ds(r, K)], idx_v)
      copies = []
      for j in range(K):
        copies.append(
            pltpu.async_copy(
                table_hbm.at[idx_v.at[j]],
                rows_v.at[pl.ds(j * 128, 128)],
                sem,
            ))
      for c in copies:
        c.wait()
      pltpu.sync_copy(rows_v, out_hbm.at[pl.ds(r * 128, K * 128)])
      return carry

    lax.fori_loop(0, n_out, body, 0)

  return gather_kernel


_gather_rays = _make_sc_gather(N_RAYS, 8, TOTAL)
_gather_emb = _make_sc_gather(NUM_EMB, 8, TOTAL * 8)

_RB = 2048
_RBF = 512


def _hash_body(rr_ref, tn_ref, tf_ref, idx_ref, frac_ref):
  rr = rr_ref[...]
  tm = 0.5 * (tn_ref[...] + tf_ref[...])
  fr = []
  x0i = []
  for d in range(3):
    xd = rr[:, d:d + 1] + tm * rr[:, d + 3:d + 4]
    xs = xd / jnp.float32(VOXEL)
    x0 = jnp.floor(xs)
    fr.append(xs - x0)
    x0i.append(x0.astype(jnp.int32))
  frac_ref[...] = jnp.concatenate(fr, axis=1)
  cols = []
  for i in (0, 1):
    for j in (0, 1):
      for k in (0, 1):
        ux = (x0i[0] + i).astype(jnp.uint32)
        uy = (x0i[1] + j).astype(jnp.uint32) * jnp.uint32(_P1)
        uz = (x0i[2] + k).astype(jnp.uint32) * jnp.uint32(_P2)
        h = (ux ^ uy ^ uz) % jnp.uint32(NUM_EMB)
        cols.append(h.astype(jnp.int32))
  idx_ref[...] = jnp.concatenate(cols, axis=1)


def _field_body(emb_ref, frac_ref, tn_ref, tf_ref, beta_ref,
                rgb_ref, grads_ref, sdt_ref):
  fx = frac_ref[0]
  fy = frac_ref[1]
  fz = frac_ref[2]
  mx = 1.0 - fx
  my = 1.0 - fy
  mz = 1.0 - fz
  sdf = jnp.zeros_like(fx)
  r0 = jnp.zeros_like(fx)
  r1 = jnp.zeros_like(fx)
  r2 = jnp.zeros_like(fx)
  gx = jnp.zeros_like(fx)
  gy = jnp.zeros_like(fx)
  gz = jnp.zeros_like(fx)
  c = 0
  for i in (0, 1):
    for j in (0, 1):
      for k in (0, 1):
        tx = fx if i else mx
        ty = fy if j else my
        tz = fz if k else mz
        tyz = ty * tz
        w = tx * tyz
        sdf_c = emb_ref[c, 0]
        sdf = sdf + w * sdf_c
        r0 = r0 + w * emb_ref[c, 1]
        r1 = r1 + w * emb_ref[c, 2]
        r2 = r2 + w * emb_ref[c, 3]
        gx = gx + sdf_c * (tyz if i else -tyz)
        txz = tx * tz
        gy = gy + sdf_c * (txz if j else -txz)
        txy = tx * ty
        gz = gz + sdf_c * (txy if k else -txy)
        c += 1
  inv_cell = 1.0 / jnp.float32(VOXEL)
  grads_ref[0] = gx * inv_cell
  grads_ref[1] = gy * inv_cell
  grads_ref[2] = gz * inv_cell
  beta_t = MIN_BETA + jnp.abs(beta_ref[0, 0])
  alpha = 1.0 / beta_t
  sig = alpha * (0.5 + 0.5 * jnp.sign(sdf) *
                 (jnp.exp(-jnp.abs(sdf) / beta_t) - 1.0))
  sdt_ref[...] = sig * (tf_ref[...] - tn_ref[...])
  rgb_ref[0] = r0
  rgb_ref[1] = r1
  rgb_ref[2] = r2


def _hash_stage(ray_rows, t_nears, t_fars):
  grid = TOTAL // _RB
  return pl.pallas_call(
      _hash_body,
      grid=(grid,),
      in_specs=[
          pl.BlockSpec((_RB, 8), lambda i: (i, 0)),
          pl.BlockSpec((_RB, 1), lambda i: (i, 0)),
          pl.BlockSpec((_RB, 1), lambda i: (i, 0)),
      ],
      out_specs=[
          pl.BlockSpec((_RB, 8), lambda i: (i, 0)),
          pl.BlockSpec((_RB, 3), lambda i: (i, 0)),
      ],
      out_shape=[
          jax.ShapeDtypeStruct((TOTAL, 8), jnp.int32),
          jax.ShapeDtypeStruct((TOTAL, 3), jnp.float32),
      ],
  )(ray_rows, t_nears, t_fars)


def _field_stage(embp, fracp, tnp, tfp, beta2d):
  tp = TOTAL // 128
  br = 256
  grid = tp // br
  return pl.pallas_call(
      _field_body,
      grid=(grid,),
      in_specs=[
          pl.BlockSpec((8, 4, br, 128), lambda i: (0, 0, i, 0)),
          pl.BlockSpec((3, br, 128), lambda i: (0, i, 0)),
          pl.BlockSpec((br, 128), lambda i: (i, 0)),
          pl.BlockSpec((br, 128), lambda i: (i, 0)),
          pl.BlockSpec((1, 1), lambda i: (0, 0)),
      ],
      out_specs=[
          pl.BlockSpec((3, br, 128), lambda i: (0, i, 0)),
          pl.BlockSpec((3, br, 128), lambda i: (0, i, 0)),
          pl.BlockSpec((br, 128), lambda i: (i, 0)),
      ],
      out_shape=[
          jax.ShapeDtypeStruct((3, tp, 128), jnp.float32),
          jax.ShapeDtypeStruct((3, tp, 128), jnp.float32),
          jax.ShapeDtypeStruct((tp, 128), jnp.float32),
      ],
  )(embp, fracp, tnp, tfp, beta2d)


@jax.jit
def _kernel_impl(rays_o, rays_d, rays_d_norm, ray_indices, t_nears, t_fars,
                 table, beta):
  rays_cat = jnp.concatenate(
      [rays_o, rays_d, jnp.zeros((N_RAYS, 2), jnp.float32)], axis=1)
  ridx2d = ray_indices.reshape(TOTAL // 128, 128)
  ray_rows = _gather_rays(rays_cat, ridx2d)

  idx, frac = _hash_stage(ray_rows, t_nears, t_fars)

  table_pad = jnp.concatenate(
      [table, jnp.zeros((NUM_EMB, 3), jnp.float32)], axis=1)
  emb_rows = _gather_emb(table_pad, idx.reshape(TOTAL * 8 // 128, 128))
  tp = TOTAL // 128
  embp = jnp.transpose(emb_rows.reshape(TOTAL, 8, 8)[:, :, :4],
                       (1, 2, 0)).reshape(8, 4, tp, 128)
  fracp = jnp.transpose(frac, (1, 0)).reshape(3, tp, 128)
  rgbp, gradp, sdtp = _field_stage(embp, fracp, t_nears.reshape(tp, 128),
                                   t_fars.reshape(tp, 128), beta.reshape(1, 1))
  rgb = jnp.transpose(rgbp.reshape(3, TOTAL), (1, 0))
  sdf_grads = jnp.transpose(gradp.reshape(3, TOTAL), (1, 0))

  gnorm = jnp.sqrt(jnp.sum(sdf_grads * sdf_grads, axis=1, keepdims=True))
  normals = sdf_grads / jnp.maximum(gnorm, 1e-12)

  sdt = sdtp.reshape(TOTAL)
  cum = jnp.cumsum(sdt)
  excl = cum - sdt
  seg_start = jnp.concatenate(
      [jnp.ones((1,), bool), ray_indices[1:] != ray_indices[:-1]])
  base = lax.cummax(jnp.where(seg_start, excl, -jnp.inf))
  trans = jnp.exp(-(excl - base))
  weights = (1.0 - jnp.exp(-sdt)) * trans

  t_mid = 0.5 * (t_nears + t_fars)
  w1 = weights[:, None]
  vals = jnp.concatenate([w1 * rgb, w1 * t_mid, w1 * normals, w1], axis=1)
  seg = jax.ops.segment_sum(vals, ray_indices, num_segments=N_RAYS,
                            indices_are_sorted=True)
  rendered_rgb = seg[:, 0:3]
  rendered_depth = seg[:, 3:4] / rays_d_norm
  rendered_normals = seg[:, 4:7]
  accumulated_weights = seg[:, 7:8]
  return (rendered_rgb, rendered_depth, rendered_normals,
          accumulated_weights, sdf_grads)


def kernel(rays_o, rays_d, rays_d_norm, ray_indices, t_nears, t_fars, table,
           beta):
  return _kernel_impl(rays_o, rays_d, rays_d_norm, ray_indices, t_nears,
                      t_fars, table, beta)


# speedup vs baseline: 19.5159x; 1.3980x over previous
import functools

import jax
import jax.numpy as jnp
from jax import lax
from jax.experimental import pallas as pl
from jax.experimental.pallas import tpu as pltpu
from jax.experimental.pallas import tpu_sc as plsc

N_RAYS = 4096
TOTAL = 131072
NUM_EMB = 120000
VOXEL = 0.02
MIN_BETA = VOXEL

_NC = 2
_NS = 16
_NW = _NC * _NS

_P1 = 2654435761
_P2 = 805459861


def _make_sc_gather(V, D, B, K=16):
  IB = B // 128
  per_w = IB // _NW
  n_out = per_w // K
  assert IB % _NW == 0 and per_w % K == 0
  mesh = plsc.VectorSubcoreMesh(core_axis_name="c", subcore_axis_name="s")

  @functools.partial(
      pl.kernel,
      mesh=mesh,
      out_type=jax.ShapeDtypeStruct((B, D), jnp.float32),
      compiler_params=pltpu.CompilerParams(use_tc_tiling_on_sc=False),
      scratch_types=[
          pltpu.VMEM((K, 128), jnp.int32),
          pltpu.VMEM((K * 128, D), jnp.float32),
          pltpu.SemaphoreType.DMA,
      ],
  )
  def gather_kernel(table_hbm, idx_hbm, out_hbm, idx_v, rows_v, sem):
    wid = lax.axis_index("s") * _NC + lax.axis_index("c")
    row0 = wid * per_w

    def body(g, carry):
      r = row0 + g * K
      pltpu.sync_copy(idx_hbm.at[pl.ds(r, K)], idx_v)
      copies = []
      for j in range(K):
        copies.append(
            pltpu.async_copy(
                table_hbm.at[idx_v.at[j]],
                rows_v.at[pl.ds(j * 128, 128)],
                sem,
            ))
      for c in copies:
        c.wait()
      pltpu.sync_copy(rows_v, out_hbm.at[pl.ds(r * 128, K * 128)])
      return carry

    lax.fori_loop(0, n_out, body, 0)

  return gather_kernel


_gather_rays = _make_sc_gather(N_RAYS, 8, TOTAL)
_gather_emb = _make_sc_gather(NUM_EMB, 8, TOTAL * 8)

_RB = 2048
_RBF = 512


def _hash_body(rays_ref, tn_ref, tf_ref, idx_ref, frac_ref):
  tm = 0.5 * (tn_ref[...] + tf_ref[...])
  x0i = []
  for d in range(3):
    xd = rays_ref[d] + tm * rays_ref[d + 3]
    xs = xd / jnp.float32(VOXEL)
    x0 = jnp.floor(xs)
    frac_ref[d] = xs - x0
    x0i.append(x0.astype(jnp.int32))
  c = 0
  for i in (0, 1):
    for j in (0, 1):
      for k in (0, 1):
        ux = (x0i[0] + i).astype(jnp.uint32)
        uy = (x0i[1] + j).astype(jnp.uint32) * jnp.uint32(_P1)
        uz = (x0i[2] + k).astype(jnp.uint32) * jnp.uint32(_P2)
        h = (ux ^ uy ^ uz) % jnp.uint32(NUM_EMB)
        idx_ref[c] = h.astype(jnp.int32)
        c += 1


def _field_body(emb_ref, frac_ref, tn_ref, tf_ref, beta_ref,
                rgb_ref, grads_ref, sdt_ref):
  fx = frac_ref[0]
  fy = frac_ref[1]
  fz = frac_ref[2]
  mx = 1.0 - fx
  my = 1.0 - fy
  mz = 1.0 - fz
  sdf = jnp.zeros_like(fx)
  r0 = jnp.zeros_like(fx)
  r1 = jnp.zeros_like(fx)
  r2 = jnp.zeros_like(fx)
  gx = jnp.zeros_like(fx)
  gy = jnp.zeros_like(fx)
  gz = jnp.zeros_like(fx)
  c = 0
  for i in (0, 1):
    for j in (0, 1):
      for k in (0, 1):
        tx = fx if i else mx
        ty = fy if j else my
        tz = fz if k else mz
        tyz = ty * tz
        w = tx * tyz
        sdf_c = emb_ref[c, 0]
        sdf = sdf + w * sdf_c
        r0 = r0 + w * emb_ref[c, 1]
        r1 = r1 + w * emb_ref[c, 2]
        r2 = r2 + w * emb_ref[c, 3]
        gx = gx + sdf_c * (tyz if i else -tyz)
        txz = tx * tz
        gy = gy + sdf_c * (txz if j else -txz)
        txy = tx * ty
        gz = gz + sdf_c * (txy if k else -txy)
        c += 1
  inv_cell = 1.0 / jnp.float32(VOXEL)
  grads_ref[0] = gx * inv_cell
  grads_ref[1] = gy * inv_cell
  grads_ref[2] = gz * inv_cell
  beta_t = MIN_BETA + jnp.abs(beta_ref[0, 0])
  alpha = 1.0 / beta_t
  sig = alpha * (0.5 + 0.5 * jnp.sign(sdf) *
                 (jnp.exp(-jnp.abs(sdf) / beta_t) - 1.0))
  sdt_ref[...] = sig * (tf_ref[...] - tn_ref[...])
  rgb_ref[0] = r0
  rgb_ref[1] = r1
  rgb_ref[2] = r2


def _hash_stage(raysp, tnp, tfp):
  tp = TOTAL // 128
  br = 256
  grid = tp // br
  return pl.pallas_call(
      _hash_body,
      grid=(grid,),
      in_specs=[
          pl.BlockSpec((6, br, 128), lambda i: (0, i, 0)),
          pl.BlockSpec((br, 128), lambda i: (i, 0)),
          pl.BlockSpec((br, 128), lambda i: (i, 0)),
      ],
      out_specs=[
          pl.BlockSpec((8, br, 128), lambda i: (0, i, 0)),
          pl.BlockSpec((3, br, 128), lambda i: (0, i, 0)),
      ],
      out_shape=[
          jax.ShapeDtypeStruct((8, tp, 128), jnp.int32),
          jax.ShapeDtypeStruct((3, tp, 128), jnp.float32),
      ],
  )(raysp, tnp, tfp)


def _field_stage(embp, fracp, tnp, tfp, beta2d):
  tp = TOTAL // 128
  br = 256
  grid = tp // br
  return pl.pallas_call(
      _field_body,
      grid=(grid,),
      in_specs=[
          pl.BlockSpec((8, 4, br, 128), lambda i: (0, 0, i, 0)),
          pl.BlockSpec((3, br, 128), lambda i: (0, i, 0)),
          pl.BlockSpec((br, 128), lambda i: (i, 0)),
          pl.BlockSpec((br, 128), lambda i: (i, 0)),
          pl.BlockSpec((1, 1), lambda i: (0, 0)),
      ],
      out_specs=[
          pl.BlockSpec((3, br, 128), lambda i: (0, i, 0)),
          pl.BlockSpec((3, br, 128), lambda i: (0, i, 0)),
          pl.BlockSpec((br, 128), lambda i: (i, 0)),
      ],
      out_shape=[
          jax.ShapeDtypeStruct((3, tp, 128), jnp.float32),
          jax.ShapeDtypeStruct((3, tp, 128), jnp.float32),
          jax.ShapeDtypeStruct((tp, 128), jnp.float32),
      ],
  )(embp, fracp, tnp, tfp, beta2d)


@jax.jit
def _kernel_impl(rays_o, rays_d, rays_d_norm, ray_indices, t_nears, t_fars,
                 table, beta):
  rays_cat = jnp.concatenate(
      [rays_o, rays_d, jnp.zeros((N_RAYS, 2), jnp.float32)], axis=1)
  ridx2d = ray_indices.reshape(TOTAL // 128, 128)
  ray_rows = _gather_rays(rays_cat, ridx2d)

  tp = TOTAL // 128
  tnp = t_nears.reshape(tp, 128)
  tfp = t_fars.reshape(tp, 128)
  raysp = jnp.transpose(ray_rows[:, :6], (1, 0)).reshape(6, tp, 128)
  idxp, fracp = _hash_stage(raysp, tnp, tfp)

  table_pad = jnp.concatenate(
      [table, jnp.zeros((NUM_EMB, 3), jnp.float32)], axis=1)
  emb_rows = _gather_emb(table_pad, idxp.reshape(TOTAL * 8 // 128, 128))
  embp = jnp.transpose(emb_rows.reshape(8, TOTAL, 8)[:, :, :4],
                       (0, 2, 1)).reshape(8, 4, tp, 128)
  rgbp, gradp, sdtp = _field_stage(embp, fracp, tnp, tfp, beta.reshape(1, 1))
  rgb = jnp.transpose(rgbp.reshape(3, TOTAL), (1, 0))
  sdf_grads = jnp.transpose(gradp.reshape(3, TOTAL), (1, 0))

  gnorm = jnp.sqrt(jnp.sum(sdf_grads * sdf_grads, axis=1, keepdims=True))
  normals = sdf_grads / jnp.maximum(gnorm, 1e-12)

  sdt = sdtp.reshape(TOTAL)
  cum = jnp.cumsum(sdt)
  excl = cum - sdt
  seg_start = jnp.concatenate(
      [jnp.ones((1,), bool), ray_indices[1:] != ray_indices[:-1]])
  base = lax.cummax(jnp.where(seg_start, excl, -jnp.inf))
  trans = jnp.exp(-(excl - base))
  weights = (1.0 - jnp.exp(-sdt)) * trans

  t_mid = 0.5 * (t_nears + t_fars)
  w1 = weights[:, None]
  vals = jnp.concatenate([w1 * rgb, w1 * t_mid, w1 * normals, w1], axis=1)
  seg = jax.ops.segment_sum(vals, ray_indices, num_segments=N_RAYS,
                            indices_are_sorted=True)
  rendered_rgb = seg[:, 0:3]
  rendered_depth = seg[:, 3:4] / rays_d_norm
  rendered_normals = seg[:, 4:7]
  accumulated_weights = seg[:, 7:8]
  return (rendered_rgb, rendered_depth, rendered_normals,
          accumulated_weights, sdf_grads)


def kernel(rays_o, rays_d, rays_d_norm, ray_indices, t_nears, t_fars, table,
           beta):
  return _kernel_impl(rays_o, rays_d, rays_d_norm, ray_indices, t_nears,
                      t_fars, table, beta)
